# baseline (speedup 1.0000x reference)
import functools

import jax
import jax.numpy as jnp
from jax import lax
from jax.experimental import pallas as pl
from jax.experimental.pallas import tpu as pltpu
from jax.experimental.pallas import tpu_sc as plsc

WIDTH = 4096
D = 2048
BATCH = 2
ROWS = BATCH * WIDTH
NC, NS = 2, 16
NW = NC * NS
RPW = ROWS // NW
CH = 8
NCHUNK = RPW // CH
G = 3
W = 3
NBUF = G + W


def _permute_body(perm_hbm, x_hbm, out_hbm, idx_v, *rest):
    bufs = list(rest[:NBUF])
    sgs = list(rest[NBUF:2 * NBUF])
    sws = list(rest[2 * NBUF:3 * NBUF])

    c = lax.axis_index("c")
    s = lax.axis_index("s")
    wid = s * NC + c
    batch = wid // (NW // BATCH)
    pidx = wid % (NW // BATCH)
    pbase = pidx * RPW
    out_base = batch * WIDTH + pbase
    off = batch * WIDTH

    pltpu.sync_copy(perm_hbm.at[pl.ds(pbase, RPW)], idx_v)
    for i in range(RPW // 16):
        idx_v[pl.ds(i * 16, 16)] = idx_v[pl.ds(i * 16, 16)] + off

    gathers = [None] * NCHUNK
    writes = [None] * NCHUNK

    def g_start(it):
        bi = it % NBUF
        gathers[it] = pltpu.async_copy(
            x_hbm.at[idx_v.at[pl.ds(it * CH, CH)]], bufs[bi], sgs[bi])

    def w_start(it):
        bi = it % NBUF
        writes[it] = pltpu.async_copy(
            bufs[bi], out_hbm.at[pl.ds(out_base + it * CH, CH)], sws[bi])

    for it in range(min(G, NCHUNK)):
        g_start(it)
    for it in range(NCHUNK):
        if it + G < NCHUNK:
            if it - W >= 0:
                writes[it - W].wait()
            g_start(it + G)
        gathers[it].wait()
        w_start(it)
    for it in range(max(0, NCHUNK - NBUF), NCHUNK):
        writes[it].wait()


@jax.jit
def _permute_flat(perm, xf):
    mesh = plsc.VectorSubcoreMesh(
        core_axis_name="c", subcore_axis_name="s",
        num_cores=NC, num_subcores=NS)
    run = pl.kernel(
        _permute_body,
        out_type=jax.ShapeDtypeStruct((ROWS, D), jnp.float32),
        mesh=mesh,
        scratch_types=(
            [pltpu.VMEM((RPW,), jnp.int32)]
            + [pltpu.VMEM((CH, D), jnp.float32) for _ in range(NBUF)]
            + [pltpu.SemaphoreType.DMA for _ in range(2 * NBUF)]
        ),
        name="sc_row_permute",
    )
    return run(perm, xf)


def kernel(X, perm):
    xf = X.reshape(ROWS, D)
    out = _permute_flat(perm, xf)
    return out.reshape(X.shape)


# speedup vs baseline: 1.0077x; 1.0077x over previous
import functools

import jax
import jax.numpy as jnp
from jax import lax
from jax.experimental import pallas as pl
from jax.experimental.pallas import tpu as pltpu
from jax.experimental.pallas import tpu_sc as plsc

WIDTH = 4096
D = 2048
BATCH = 2
ROWS = BATCH * WIDTH
NC, NS = 2, 16
NW = NC * NS
RPW = ROWS // NW
CH = 8
NCHUNK = RPW // CH
G = 4
W = 3
NBUF = G + W


def _permute_body(perm_hbm, x_hbm, out_hbm, idx_v, *rest):
    bufs = list(rest[:NBUF])
    sgs = list(rest[NBUF:2 * NBUF])
    sws = list(rest[2 * NBUF:3 * NBUF])

    c = lax.axis_index("c")
    s = lax.axis_index("s")
    wid = s * NC + c
    batch = wid // (NW // BATCH)
    pidx = wid % (NW // BATCH)
    pbase = pidx * RPW
    out_base = batch * WIDTH + pbase
    off = batch * WIDTH

    pltpu.sync_copy(perm_hbm.at[pl.ds(pbase, RPW)], idx_v)
    for i in range(RPW // 16):
        idx_v[pl.ds(i * 16, 16)] = idx_v[pl.ds(i * 16, 16)] + off

    gathers = [None] * NCHUNK
    writes = [None] * NCHUNK

    def g_start(it):
        bi = it % NBUF
        gathers[it] = pltpu.async_copy(
            x_hbm.at[idx_v.at[pl.ds(it * CH, CH)]], bufs[bi], sgs[bi])

    def w_start(it):
        bi = it % NBUF
        writes[it] = pltpu.async_copy(
            bufs[bi], out_hbm.at[pl.ds(out_base + it * CH, CH)], sws[bi])

    for it in range(min(G, NCHUNK)):
        g_start(it)
    for it in range(NCHUNK):
        if it + G < NCHUNK:
            if it - W >= 0:
                writes[it - W].wait()
            g_start(it + G)
        gathers[it].wait()
        w_start(it)
    for it in range(max(0, NCHUNK - NBUF), NCHUNK):
        writes[it].wait()


@jax.jit
def _permute_flat(perm, xf):
    mesh = plsc.VectorSubcoreMesh(
        core_axis_name="c", subcore_axis_name="s",
        num_cores=NC, num_subcores=NS)
    run = pl.kernel(
        _permute_body,
        out_type=jax.ShapeDtypeStruct((ROWS, D), jnp.float32),
        mesh=mesh,
        scratch_types=(
            [pltpu.VMEM((RPW,), jnp.int32)]
            + [pltpu.VMEM((CH, D), jnp.float32) for _ in range(NBUF)]
            + [pltpu.SemaphoreType.DMA for _ in range(2 * NBUF)]
        ),
        name="sc_row_permute",
    )
    return run(perm, xf)


def kernel(X, perm):
    xf = X.reshape(ROWS, D)
    out = _permute_flat(perm, xf)
    return out.reshape(X.shape)
